# baseline (speedup 1.0000x reference)
import functools

import jax
import jax.numpy as jnp
from jax import lax
from jax.experimental import pallas as pl
from jax.experimental.pallas import tpu as pltpu
from jax.experimental.pallas import tpu_sc as plsc

_EMBED = 32
_MAXLEN = 200
_NC = 2
_NS = 16
_NW = _NC * _NS
_CH = 40
_CPS = _MAXLEN // _CH


def _sc_body(x_hbm, tok_hbm, pos_hbm, out_hbm, idx_v, pos_v, seq_v, sem):
    n_seq = x_hbm.shape[0]
    spw = n_seq // _NW
    wid = lax.axis_index("s") * _NC + lax.axis_index("c")

    pltpu.sync_copy(x_hbm.at[pl.ds(wid * spw, spw)], idx_v)
    pltpu.sync_copy(pos_hbm, pos_v)

    def seq_body(s, carry):
        handles = [
            pltpu.async_copy(
                tok_hbm.at[idx_v.at[s, k]],
                seq_v.at[pl.ds(k * _CH, _CH)],
                sem,
            )
            for k in range(_CPS)
        ]
        for h in handles:
            h.wait()

        def row_body(r, c):
            plsc.addupdate(seq_v.at[r, pl.ds(0, 16)], pos_v[r, pl.ds(0, 16)])
            plsc.addupdate(seq_v.at[r, pl.ds(16, 16)], pos_v[r, pl.ds(16, 16)])
            return c

        lax.fori_loop(0, _MAXLEN, row_body, 0)

        base = (wid * spw + s) * _MAXLEN
        pltpu.sync_copy(seq_v, out_hbm.at[pl.ds(base, _MAXLEN)])
        return carry

    lax.fori_loop(0, spw, seq_body, 0)


@jax.jit
def _sc_embed(x_idx, token_table, pos_table):
    n_seq = x_idx.shape[0]
    mesh = plsc.VectorSubcoreMesh(core_axis_name="c", subcore_axis_name="s")
    spw = n_seq // _NW
    return pl.kernel(
        _sc_body,
        out_type=jax.ShapeDtypeStruct((n_seq * _MAXLEN, _EMBED), jnp.float32),
        mesh=mesh,
        scratch_types=[
            pltpu.VMEM((spw, _CPS, _CH), jnp.int32),
            pltpu.VMEM((_MAXLEN, _EMBED), jnp.float32),
            pltpu.VMEM((_MAXLEN, _EMBED), jnp.float32),
            pltpu.SemaphoreType.DMA,
        ],
        compiler_params=pltpu.CompilerParams(use_tc_tiling_on_sc=False),
    )(x_idx, token_table, pos_table)


def kernel(x, token_table, pos_table):
    batch, maxlen = x.shape
    x_idx = x.astype(jnp.int32).reshape(batch, _CPS, _CH)
    out = _sc_embed(x_idx, token_table, pos_table)
    return out.reshape(batch, maxlen, _EMBED)
